# baseline (speedup 1.0000x reference)
import functools

import jax
import jax.numpy as jnp
from jax import lax
from jax.experimental import pallas as pl
from jax.experimental.pallas import tpu as pltpu
from jax.experimental.pallas import tpu_sc as plsc

HIDDEN = 256
N_NODES = 100000

NC = 2
NS = 16
NW = NC * NS
LANES = 16
HBLK = HIDDEN // LANES

GPC = 2
CHUNK = GPC * LANES
NCHUNK = 98
B_PER_W = CHUNK * NCHUNK
LAST_W_ROWS = N_NODES - (NW - 1) * B_PER_W
NCHUNK_LAST = LAST_W_ROWS // CHUNK
NBUF = 4
PRE_CHUNKS = 8
PRE_ROWS = PRE_CHUNKS * CHUNK

_mesh = plsc.VectorSubcoreMesh(core_axis_name="c", subcore_axis_name="s")


@functools.partial(
    pl.kernel,
    out_type=jax.ShapeDtypeStruct((N_NODES, HIDDEN), jnp.float32),
    mesh=_mesh,
    scratch_types=[
        pltpu.VMEM((B_PER_W,), jnp.int32),
        pltpu.VMEM((2, HIDDEN), jnp.float32),
        pltpu.VMEM((NBUF, CHUNK, HIDDEN), jnp.float32),
        pltpu.SemaphoreType.DMA,
        pltpu.SemaphoreType.DMA,
        pltpu.SemaphoreType.DMA,
    ],
)
def _sc_lookup(w_hbm, idx_hbm, out_hbm, idx_v, w_v, rows_v, sem, sem_a, sem_b):
    wid = lax.axis_index("s") * NC + lax.axis_index("c")
    base = wid * B_PER_W
    pltpu.async_copy(
        idx_hbm.at[pl.ds(base, PRE_ROWS)], idx_v.at[pl.ds(0, PRE_ROWS)], sem_a
    )

    @pl.when(wid < NW - 1)
    def _():
        pltpu.async_copy(
            idx_hbm.at[pl.ds(base + PRE_ROWS, B_PER_W - PRE_ROWS)],
            idx_v.at[pl.ds(PRE_ROWS, B_PER_W - PRE_ROWS)],
            sem_b,
        )

    @pl.when(wid == NW - 1)
    def _():
        pltpu.async_copy(
            idx_hbm.at[pl.ds(base + PRE_ROWS, LAST_W_ROWS - PRE_ROWS)],
            idx_v.at[pl.ds(PRE_ROWS, LAST_W_ROWS - PRE_ROWS)],
            sem_b,
        )

    pltpu.sync_copy(w_hbm, w_v)

    w0 = [w_v[0, pl.ds(h * LANES, LANES)] for h in range(HBLK)]
    w1 = [w_v[1, pl.ds(h * LANES, LANES)] for h in range(HBLK)]

    def out_slice(j):
        return out_hbm.at[pl.ds(base + j * CHUNK, CHUNK)]

    def compute_chunk(j, buf):
        rows = rows_v.at[buf]

        def group(g2, carry):
            v = idx_v[pl.ds((j * GPC + g2) * LANES, LANES)]
            for r in range(LANES):
                pred = v[r] == 0
                for h in range(HBLK):
                    rows[g2 * LANES + r, pl.ds(h * LANES, LANES)] = (
                        jnp.where(pred, w0[h], w1[h])
                    )
            return carry

        lax.fori_loop(0, GPC, group, 0)

    def body(j, carry):
        buf = lax.rem(j, NBUF)

        @pl.when(j == 0)
        def _():
            pltpu.make_async_copy(
                idx_hbm.at[pl.ds(base, PRE_ROWS)],
                idx_v.at[pl.ds(0, PRE_ROWS)],
                sem_a,
            ).wait()

        @pl.when(j == PRE_CHUNKS)
        def _():
            @pl.when(wid < NW - 1)
            def _():
                pltpu.make_async_copy(
                    idx_hbm.at[pl.ds(base + PRE_ROWS, B_PER_W - PRE_ROWS)],
                    idx_v.at[pl.ds(PRE_ROWS, B_PER_W - PRE_ROWS)],
                    sem_b,
                ).wait()

            @pl.when(wid == NW - 1)
            def _():
                pltpu.make_async_copy(
                    idx_hbm.at[pl.ds(base + PRE_ROWS, LAST_W_ROWS - PRE_ROWS)],
                    idx_v.at[pl.ds(PRE_ROWS, LAST_W_ROWS - PRE_ROWS)],
                    sem_b,
                ).wait()

        @pl.when(j >= NBUF)
        def _():
            pltpu.make_async_copy(rows_v.at[buf], out_slice(j - NBUF), sem).wait()

        compute_chunk(j, buf)
        pltpu.async_copy(rows_v.at[buf], out_slice(j), sem)
        return carry

    nchunk = jnp.where(wid == NW - 1, NCHUNK_LAST, NCHUNK)
    lax.fori_loop(0, nchunk, body, 0)

    def drain(k, carry):
        j = nchunk - NBUF + k
        pltpu.make_async_copy(
            rows_v.at[lax.rem(j, NBUF)], out_slice(j), sem
        ).wait()
        return carry

    lax.fori_loop(0, NBUF, drain, 0)


def kernel(x, W):
    return _sc_lookup(W, x.astype(jnp.int32))


# speedup vs baseline: 1.0170x; 1.0170x over previous
import functools

import jax
import jax.numpy as jnp
from jax import lax
from jax.experimental import pallas as pl
from jax.experimental.pallas import tpu as pltpu
from jax.experimental.pallas import tpu_sc as plsc

HIDDEN = 256
N_NODES = 100000

NC = 2
NS = 16
NW = NC * NS
LANES = 16
HBLK = HIDDEN // LANES

GPC = 2
CHUNK = GPC * LANES
NCHUNK = 98
B_PER_W = CHUNK * NCHUNK
LAST_W_ROWS = N_NODES - (NW - 1) * B_PER_W
NCHUNK_LAST = LAST_W_ROWS // CHUNK
NBUF = 4

_mesh = plsc.VectorSubcoreMesh(core_axis_name="c", subcore_axis_name="s")


@functools.partial(
    pl.kernel,
    out_type=jax.ShapeDtypeStruct((N_NODES, HIDDEN), jnp.float32),
    mesh=_mesh,
    scratch_types=[
        pltpu.VMEM((B_PER_W,), jnp.int32),
        pltpu.VMEM((2, HIDDEN), jnp.float32),
        pltpu.VMEM((NBUF, CHUNK, HIDDEN), jnp.float32),
        pltpu.SemaphoreType.DMA,
    ],
)
def _sc_lookup(w_hbm, idx_hbm, out_hbm, idx_v, w_v, rows_v, sem):
    wid = lax.axis_index("s") * NC + lax.axis_index("c")
    base = wid * B_PER_W
    @pl.when(wid < NW - 1)
    def _():
        pltpu.sync_copy(idx_hbm.at[pl.ds(base, B_PER_W)], idx_v)

    @pl.when(wid == NW - 1)
    def _():
        pltpu.sync_copy(
            idx_hbm.at[pl.ds(base, LAST_W_ROWS)],
            idx_v.at[pl.ds(0, LAST_W_ROWS)],
        )

    pltpu.sync_copy(w_hbm, w_v)

    w0 = [w_v[0, pl.ds(h * LANES, LANES)] for h in range(HBLK)]
    w1 = [w_v[1, pl.ds(h * LANES, LANES)] for h in range(HBLK)]

    def out_slice(j):
        return out_hbm.at[pl.ds(base + j * CHUNK, CHUNK)]

    def compute_chunk(j, buf):
        rows = rows_v.at[buf]

        def group(g2, carry):
            v = idx_v[pl.ds((j * GPC + g2) * LANES, LANES)]
            for r in range(LANES):
                pred = v[r] == 0
                for h in range(HBLK):
                    rows[g2 * LANES + r, pl.ds(h * LANES, LANES)] = (
                        jnp.where(pred, w0[h], w1[h])
                    )
            return carry

        lax.fori_loop(0, GPC, group, 0)

    def body(j, carry):
        buf = lax.rem(j, NBUF)

        @pl.when(j >= NBUF)
        def _():
            pltpu.make_async_copy(rows_v.at[buf], out_slice(j - NBUF), sem).wait()

        compute_chunk(j, buf)
        pltpu.async_copy(rows_v.at[buf], out_slice(j), sem)
        return carry

    nchunk = jnp.where(wid == NW - 1, NCHUNK_LAST, NCHUNK)
    lax.fori_loop(0, nchunk, body, 0)

    def drain(k, carry):
        j = nchunk - NBUF + k
        pltpu.make_async_copy(
            rows_v.at[lax.rem(j, NBUF)], out_slice(j), sem
        ).wait()
        return carry

    lax.fori_loop(0, NBUF, drain, 0)


def kernel(x, W):
    return _sc_lookup(W, x.astype(jnp.int32))
